# baseline (speedup 1.0000x reference)
import functools

import jax
import jax.numpy as jnp
from jax import lax
from jax.experimental import pallas as pl
from jax.experimental.pallas import tpu as pltpu
from jax.experimental.pallas import tpu_sc as plsc

N_NODES = 10000
N_EDGES = 320000
D_IN = 128
H1 = 128
H2 = 64
H3 = 32
N_OUT = 2

NC = 2
NS = 16
NW = NC * NS

CHUNK = 128
K = 80
G = 8
NG = K // G
E_PAD = NW * K * CHUNK
N_ACC = 10240

ROW_BLK = 10240
GRID = N_ACC // ROW_BLK
CW = 16


def _build_seg_mean(d, with_cnt):
    rows_per_tec = N_ACC // NS
    preload = d <= 64
    chunk = CHUNK
    k = E_PAD // (NW * chunk)
    nbuf = 6 if preload else 2
    gu = 16 if preload else 8
    ng = k // gu
    gs = k if preload else 8
    n_init = rows_per_tec // chunk

    out_type = [jax.ShapeDtypeStruct((NC, N_ACC, d), jnp.float32)]
    scratch = [
        pltpu.VMEM((gs, chunk), jnp.int32),
        pltpu.VMEM((gs, chunk), jnp.int32),
        pltpu.VMEM((nbuf, chunk, d), jnp.float32),
        pltpu.VMEM_SHARED((N_ACC, d), jnp.float32),
    ] + [pltpu.SemaphoreType.DMA] * (2 * nbuf)
    if with_cnt:
        out_type.append(jax.ShapeDtypeStruct((NC, N_ACC, CW), jnp.float32))
        scratch += [
            pltpu.VMEM((chunk, CW), jnp.float32),
            pltpu.VMEM_SHARED((N_ACC, CW), jnp.float32),
            pltpu.SemaphoreType.DMA,
        ]

    def body(y_hbm, src_hbm, dst_hbm, *refs):
        i = 1
        acc_out = refs[0]
        if with_cnt:
            cnt_out = refs[i]
            i += 1
        src_v, dst_v, rows_v, acc_sh = refs[i:i + 4]
        i += 4
        gsem = refs[i:i + nbuf]
        ssem = refs[i + nbuf:i + 2 * nbuf]
        i += 2 * nbuf
        if with_cnt:
            ones_v, cnt_sh, csem = refs[i:i + 3]
        c = lax.axis_index("c")
        s = lax.axis_index("s")
        wid = c * NS + s
        base = s * rows_per_tec

        zero16 = jnp.zeros((16,), jnp.float32)

        def zrow(i, carry):
            for k2 in range(d // 16):
                rows_v[0, i, pl.ds(k2 * 16, 16)] = zero16
            if with_cnt:
                ones_v[i, pl.ds(0, 16)] = zero16
            return carry

        lax.fori_loop(0, chunk, zrow, 0)
        for r in range(n_init):
            pltpu.sync_copy(rows_v.at[0],
                            acc_sh.at[pl.ds(base + r * chunk, chunk)])
            if with_cnt:
                pltpu.sync_copy(ones_v,
                                cnt_sh.at[pl.ds(base + r * chunk, chunk)])
        if with_cnt:
            one16 = jnp.ones((16,), jnp.float32)

            def orow(i, carry):
                ones_v[i, pl.ds(0, 16)] = one16
                return carry

            lax.fori_loop(0, chunk, orow, 0)

        if preload:
            pltpu.sync_copy(src_hbm.at[wid], src_v)
            pltpu.sync_copy(dst_hbm.at[wid], dst_v)
        plsc.subcore_barrier()

        def group(g, carry):
            if preload:
                def sidx(j):
                    return src_v.at[g * gu + j]

                def didx(j):
                    return dst_v.at[g * gu + j]
            else:
                pltpu.sync_copy(src_hbm.at[wid].at[pl.ds(g * gu, gu)], src_v)
                pltpu.sync_copy(dst_hbm.at[wid].at[pl.ds(g * gu, gu)], dst_v)

                def sidx(j):
                    return src_v.at[j]

                def didx(j):
                    return dst_v.at[j]

            g_cp = [None] * gu
            s_cp = [None] * gu
            c_cp = []
            for p in range(nbuf - 1):
                g_cp[p] = pltpu.async_copy(y_hbm.at[sidx(p)], rows_v.at[p],
                                           gsem[p])
            for j in range(gu):
                b = j % nbuf
                a = j + nbuf - 1
                if a < gu:
                    if j >= 1:
                        s_cp[j - 1].wait()
                    g_cp[a] = pltpu.async_copy(
                        y_hbm.at[sidx(a)], rows_v.at[a % nbuf], gsem[a % nbuf])
                g_cp[j].wait()
                s_cp[j] = pltpu.async_copy(
                    rows_v.at[b], acc_sh.at[didx(j)], ssem[b], add=True)
                if with_cnt:
                    c_cp.append(pltpu.async_copy(
                        ones_v, cnt_sh.at[didx(j)], csem, add=True))
            for j in range(max(0, gu - nbuf), gu):
                s_cp[j].wait()
            for cp in c_cp:
                cp.wait()
            return carry

        lax.fori_loop(0, ng, group, 0)
        plsc.subcore_barrier()

        pltpu.sync_copy(acc_sh.at[pl.ds(base, rows_per_tec)],
                        acc_out.at[c].at[pl.ds(base, rows_per_tec)])
        if with_cnt:
            pltpu.sync_copy(cnt_sh.at[pl.ds(base, rows_per_tec)],
                            cnt_out.at[c].at[pl.ds(base, rows_per_tec)])

    mesh = plsc.VectorSubcoreMesh(core_axis_name="c", subcore_axis_name="s",
                                  num_cores=NC, num_subcores=NS)
    inner = pl.kernel(
        body, out_type=out_type, mesh=mesh, scratch_types=scratch,
        compiler_params=pltpu.CompilerParams(use_tc_tiling_on_sc=False))

    def call(y, src_flat, dst_flat):
        return inner(y, src_flat.reshape(NW, k, chunk),
                     dst_flat.reshape(NW, k, chunk))

    return call


def _dot_t(a, w):
    return lax.dot_general(a, w, (((1,), (1,)), ((), ())),
                           preferred_element_type=jnp.float32)


def _pre1_body(x_ref, wl_ref, bl_ref, wr_ref, y_ref, s_ref):
    xb = x_ref[...]
    y_ref[...] = _dot_t(xb, wl_ref[...])
    s_ref[...] = _dot_t(xb, wr_ref[...]) + bl_ref[...]


def _combine(a0_ref, a1_ref, c0_ref, c1_ref, sp_ref):
    cnt = c0_ref[0, :, 0:1] + c1_ref[0, :, 0:1]
    inv = 1.0 / jnp.maximum(cnt, 1.0)
    return jnp.maximum((a0_ref[0] + a1_ref[0]) * inv + sp_ref[...], 0.0)


def _mid_body(a0_ref, a1_ref, c0_ref, c1_ref, sp_ref, wl_ref, bl_ref, wr_ref,
              y_ref, s_ref):
    h = _combine(a0_ref, a1_ref, c0_ref, c1_ref, sp_ref)
    y_ref[...] = _dot_t(h, wl_ref[...])
    s_ref[...] = _dot_t(h, wr_ref[...]) + bl_ref[...]


def _final_body(a0_ref, a1_ref, c0_ref, c1_ref, sp_ref, wc_ref, bc_ref,
                out_ref):
    h = _combine(a0_ref, a1_ref, c0_ref, c1_ref, sp_ref)
    out_ref[...] = _dot_t(h, wc_ref[...]) + bc_ref[...]


def _rows(d):
    return pl.BlockSpec((ROW_BLK, d), lambda i: (i, 0))


def _full(shape):
    nd = len(shape)
    return pl.BlockSpec(shape, lambda i, _n=nd: (0,) * _n)


def _part(d, which):
    return pl.BlockSpec((1, ROW_BLK, d), lambda i, _w=which: (_w, i, 0))


def _build_pre1():
    return pl.pallas_call(
        _pre1_body,
        grid=(GRID,),
        in_specs=[_rows(D_IN), _full((H1, D_IN)), _full((1, H1)),
                  _full((H1, D_IN))],
        out_specs=[_rows(H1), _rows(H1)],
        out_shape=[jax.ShapeDtypeStruct((N_NODES, H1), jnp.float32)] * 2,
    )


def _build_mid(d_in, d_out):
    return pl.pallas_call(
        _mid_body,
        grid=(GRID,),
        in_specs=[_part(d_in, 0), _part(d_in, 1), _part(16, 0), _part(16, 1),
                  _rows(d_in), _full((d_out, d_in)), _full((1, d_out)),
                  _full((d_out, d_in))],
        out_specs=[_rows(d_out), _rows(d_out)],
        out_shape=[jax.ShapeDtypeStruct((N_NODES, d_out), jnp.float32)] * 2,
    )


def _build_final():
    return pl.pallas_call(
        _final_body,
        grid=(GRID,),
        in_specs=[_part(H3, 0), _part(H3, 1), _part(16, 0), _part(16, 1),
                  _rows(H3), _full((N_OUT, H3)), _full((1, N_OUT))],
        out_specs=_rows(N_OUT),
        out_shape=jax.ShapeDtypeStruct((N_NODES, N_OUT), jnp.float32),
    )


_pre1 = _build_pre1()
_mid2 = _build_mid(H1, H2)
_mid3 = _build_mid(H2, H3)
_final = _build_final()

_SEG_CACHE = {}


def _seg_mean(d, with_cnt):
    key = (d, with_cnt)
    if key not in _SEG_CACHE:
        _SEG_CACHE[key] = _build_seg_mean(d, with_cnt)
    return _SEG_CACHE[key]


def kernel(x, edge_index, Wl1, bl1, Wr1, Wl2, bl2, Wr2, Wl3, bl3, Wr3, Wc, bc):
    src = edge_index[0].astype(jnp.int32)
    dst = edge_index[1].astype(jnp.int32)
    pad = E_PAD - N_EDGES
    pad_src = jnp.arange(pad, dtype=jnp.int32) % N_NODES
    pad_dst = N_NODES + jnp.arange(pad, dtype=jnp.int32) % (N_ACC - N_NODES)
    src = jnp.concatenate([src, pad_src])
    dst = jnp.concatenate([dst, pad_dst])

    y1, s1 = _pre1(x, Wl1, bl1.reshape(1, -1), Wr1)
    a1, cnt = _seg_mean(H1, True)(y1, src, dst)
    y2, s2 = _mid2(a1, a1, cnt, cnt, s1, Wl2, bl2.reshape(1, -1), Wr2)
    (a2,) = _seg_mean(H2, False)(y2, src, dst)
    y3, s3 = _mid3(a2, a2, cnt, cnt, s2, Wl3, bl3.reshape(1, -1), Wr3)
    (a3,) = _seg_mean(H3, False)(y3, src, dst)
    out = _final(a3, a3, cnt, cnt, s3, Wc, bc.reshape(1, -1))
    return out


# speedup vs baseline: 1.0100x; 1.0100x over previous
import jax
import jax.numpy as jnp
from jax import lax
from jax.experimental import pallas as pl
from jax.experimental.pallas import tpu as pltpu
from jax.experimental.pallas import tpu_sc as plsc

N_NODES = 10000
N_EDGES = 320000
D_IN = 128
H1 = 128
H2 = 64
H3 = 32
N_OUT = 2

NC = 2
NS = 16
NW = NC * NS

CHUNK = 128
K = 80
E_PAD = NW * K * CHUNK
N_ACC = 10240

ROW_BLK = 5120
GRID = N_ACC // ROW_BLK
CW = 16


def _build_seg_mean(d, with_cnt):
    rows_per_tec = N_ACC // NS
    preload = d <= 64
    chunk = CHUNK
    k = E_PAD // (NW * chunk)
    nbuf = 6 if preload else 2
    gu = 16 if preload else 8
    ng = k // gu
    gs = k if preload else 8
    n_init = rows_per_tec // chunk

    out_type = [jax.ShapeDtypeStruct((NC, N_ACC, d), jnp.float32)]
    scratch = [
        pltpu.VMEM((gs, chunk), jnp.int32),
        pltpu.VMEM((gs, chunk), jnp.int32),
        pltpu.VMEM((nbuf, chunk, d), jnp.float32),
        pltpu.VMEM_SHARED((N_ACC, d), jnp.float32),
    ] + [pltpu.SemaphoreType.DMA] * (2 * nbuf)
    if with_cnt:
        out_type.append(jax.ShapeDtypeStruct((NC, N_ACC, CW), jnp.float32))
        scratch += [
            pltpu.VMEM((chunk, CW), jnp.float32),
            pltpu.VMEM_SHARED((N_ACC, CW), jnp.float32),
            pltpu.SemaphoreType.DMA,
        ]

    def body(y_hbm, src_hbm, dst_hbm, *refs):
        i = 1
        acc_out = refs[0]
        if with_cnt:
            cnt_out = refs[i]
            i += 1
        src_v, dst_v, rows_v, acc_sh = refs[i:i + 4]
        i += 4
        gsem = refs[i:i + nbuf]
        ssem = refs[i + nbuf:i + 2 * nbuf]
        i += 2 * nbuf
        if with_cnt:
            ones_v, cnt_sh, csem = refs[i:i + 3]
        c = lax.axis_index("c")
        s = lax.axis_index("s")
        wid = c * NS + s
        base = s * rows_per_tec

        zero16 = jnp.zeros((16,), jnp.float32)

        def zrow(i, carry):
            for k2 in range(d // 16):
                rows_v[0, i, pl.ds(k2 * 16, 16)] = zero16
            if with_cnt:
                ones_v[i, pl.ds(0, 16)] = zero16
            return carry

        lax.fori_loop(0, chunk, zrow, 0)
        for r in range(n_init):
            pltpu.sync_copy(rows_v.at[0],
                            acc_sh.at[pl.ds(base + r * chunk, chunk)])
            if with_cnt:
                pltpu.sync_copy(ones_v,
                                cnt_sh.at[pl.ds(base + r * chunk, chunk)])
        if with_cnt:
            one16 = jnp.ones((16,), jnp.float32)

            def orow(i, carry):
                ones_v[i, pl.ds(0, 16)] = one16
                return carry

            lax.fori_loop(0, chunk, orow, 0)

        if preload:
            pltpu.sync_copy(src_hbm.at[wid], src_v)
            pltpu.sync_copy(dst_hbm.at[wid], dst_v)
        plsc.subcore_barrier()

        def group(g, carry):
            if preload:
                def sidx(j):
                    return src_v.at[g * gu + j]

                def didx(j):
                    return dst_v.at[g * gu + j]
            else:
                pltpu.sync_copy(src_hbm.at[wid].at[pl.ds(g * gu, gu)], src_v)
                pltpu.sync_copy(dst_hbm.at[wid].at[pl.ds(g * gu, gu)], dst_v)

                def sidx(j):
                    return src_v.at[j]

                def didx(j):
                    return dst_v.at[j]

            g_cp = [None] * gu
            s_cp = [None] * gu
            c_cp = []
            for p in range(nbuf - 1):
                g_cp[p] = pltpu.async_copy(y_hbm.at[sidx(p)], rows_v.at[p],
                                           gsem[p])
            for j in range(gu):
                b = j % nbuf
                a = j + nbuf - 1
                if a < gu:
                    if j >= 1:
                        s_cp[j - 1].wait()
                    g_cp[a] = pltpu.async_copy(
                        y_hbm.at[sidx(a)], rows_v.at[a % nbuf], gsem[a % nbuf])
                g_cp[j].wait()
                s_cp[j] = pltpu.async_copy(
                    rows_v.at[b], acc_sh.at[didx(j)], ssem[b], add=True)
                if with_cnt:
                    c_cp.append(pltpu.async_copy(
                        ones_v, cnt_sh.at[didx(j)], csem, add=True))
            for j in range(max(0, gu - nbuf), gu):
                s_cp[j].wait()
            for cp in c_cp:
                cp.wait()
            return carry

        lax.fori_loop(0, ng, group, 0)
        plsc.subcore_barrier()

        pltpu.sync_copy(acc_sh.at[pl.ds(base, rows_per_tec)],
                        acc_out.at[c].at[pl.ds(base, rows_per_tec)])
        if with_cnt:
            pltpu.sync_copy(cnt_sh.at[pl.ds(base, rows_per_tec)],
                            cnt_out.at[c].at[pl.ds(base, rows_per_tec)])

    mesh = plsc.VectorSubcoreMesh(core_axis_name="c", subcore_axis_name="s",
                                  num_cores=NC, num_subcores=NS)
    inner = pl.kernel(
        body, out_type=out_type, mesh=mesh, scratch_types=scratch,
        compiler_params=pltpu.CompilerParams(use_tc_tiling_on_sc=False))

    def call(y, src_flat, dst_flat):
        return inner(y, src_flat.reshape(NW, k, chunk),
                     dst_flat.reshape(NW, k, chunk))

    return call


def _dot_t(a, w):
    return lax.dot_general(a, w, (((1,), (1,)), ((), ())),
                           preferred_element_type=jnp.float32)


def _pre1_body(x_ref, wl_ref, bl_ref, wr_ref, y_ref, s_ref):
    xb = x_ref[...]
    y_ref[...] = _dot_t(xb, wl_ref[...])
    s_ref[...] = _dot_t(xb, wr_ref[...]) + bl_ref[...]


def _combine(a0_ref, a1_ref, c0_ref, c1_ref, sp_ref):
    cnt = c0_ref[0, :, 0:1] + c1_ref[0, :, 0:1]
    inv = 1.0 / jnp.maximum(cnt, 1.0)
    return jnp.maximum((a0_ref[0] + a1_ref[0]) * inv + sp_ref[...], 0.0)


def _mid_body(a0_ref, a1_ref, c0_ref, c1_ref, sp_ref, wl_ref, bl_ref, wr_ref,
              y_ref, s_ref):
    h = _combine(a0_ref, a1_ref, c0_ref, c1_ref, sp_ref)
    y_ref[...] = _dot_t(h, wl_ref[...])
    s_ref[...] = _dot_t(h, wr_ref[...]) + bl_ref[...]


def _final_body(a0_ref, a1_ref, c0_ref, c1_ref, sp_ref, wc_ref, bc_ref,
                out_ref):
    h = _combine(a0_ref, a1_ref, c0_ref, c1_ref, sp_ref)
    out_ref[...] = _dot_t(h, wc_ref[...]) + bc_ref[...]


def _rows(d):
    return pl.BlockSpec((ROW_BLK, d), lambda i: (i, 0))


def _full(shape):
    nd = len(shape)
    return pl.BlockSpec(shape, lambda i, _n=nd: (0,) * _n)


def _part(d, which):
    return pl.BlockSpec((1, ROW_BLK, d), lambda i, _w=which: (_w, i, 0))


def _build_pre1():
    return pl.pallas_call(
        _pre1_body,
        grid=(GRID,),
        in_specs=[_rows(D_IN), _full((H1, D_IN)), _full((1, H1)),
                  _full((H1, D_IN))],
        out_specs=[_rows(H1), _rows(H1)],
        out_shape=[jax.ShapeDtypeStruct((N_NODES, H1), jnp.float32)] * 2,
    )


def _build_mid(d_in, d_out):
    return pl.pallas_call(
        _mid_body,
        grid=(GRID,),
        in_specs=[_part(d_in, 0), _part(d_in, 1), _part(16, 0), _part(16, 1),
                  _rows(d_in), _full((d_out, d_in)), _full((1, d_out)),
                  _full((d_out, d_in))],
        out_specs=[_rows(d_out), _rows(d_out)],
        out_shape=[jax.ShapeDtypeStruct((N_NODES, d_out), jnp.float32)] * 2,
    )


def _build_final():
    return pl.pallas_call(
        _final_body,
        grid=(GRID,),
        in_specs=[_part(H3, 0), _part(H3, 1), _part(16, 0), _part(16, 1),
                  _rows(H3), _full((N_OUT, H3)), _full((1, N_OUT))],
        out_specs=_rows(N_OUT),
        out_shape=jax.ShapeDtypeStruct((N_NODES, N_OUT), jnp.float32),
    )


_pre1 = _build_pre1()
_mid2 = _build_mid(H1, H2)
_mid3 = _build_mid(H2, H3)
_final = _build_final()

_SEG_CACHE = {}


def _seg_mean(d, with_cnt):
    key = (d, with_cnt)
    if key not in _SEG_CACHE:
        _SEG_CACHE[key] = _build_seg_mean(d, with_cnt)
    return _SEG_CACHE[key]


def kernel(x, edge_index, Wl1, bl1, Wr1, Wl2, bl2, Wr2, Wl3, bl3, Wr3, Wc, bc):
    src = edge_index[0].astype(jnp.int32)
    dst = edge_index[1].astype(jnp.int32)
    pad = E_PAD - N_EDGES
    pad_src = jnp.arange(pad, dtype=jnp.int32) % N_NODES
    pad_dst = N_NODES + jnp.arange(pad, dtype=jnp.int32) % (N_ACC - N_NODES)
    src = jnp.concatenate([src, pad_src])
    dst = jnp.concatenate([dst, pad_dst])

    y1, s1 = _pre1(x, Wl1, bl1.reshape(1, -1), Wr1)
    a1, cnt = _seg_mean(H1, True)(y1, src, dst)
    y2, s2 = _mid2(a1, a1, cnt, cnt, s1, Wl2, bl2.reshape(1, -1), Wr2)
    (a2,) = _seg_mean(H2, False)(y2, src, dst)
    y3, s3 = _mid3(a2, a2, cnt, cnt, s2, Wl3, bl3.reshape(1, -1), Wr3)
    (a3,) = _seg_mean(H3, False)(y3, src, dst)
    out = _final(a3, a3, cnt, cnt, s3, Wc, bc.reshape(1, -1))
    return out
